# baseline (speedup 1.0000x reference)
import functools

import jax
import jax.numpy as jnp
from jax import lax
from jax.experimental import pallas as pl
from jax.experimental.pallas import tpu as pltpu
from jax.experimental.pallas import tpu_sc as plsc

_N = 10000
_NP = 10240
_C = 128
_E = 320000
_NC = 2
_NS = 16
_NW = _NC * _NS
_EPT = _E // _NW
_CHUNK = 80
_NCHUNK = _EPT // _CHUNK
_RPT = _NP // _NS
_ZR = 64


def _sc_agg_body(x_hbm, srcf_hbm, dsts_hbm, agg_out,
                 agg_sh, src_v, dst_v, rows_a, rows_b,
                 sem_a, sem_b, sem_sa, sem_sb):
    c = lax.axis_index("c")
    s = lax.axis_index("s")
    wid = c * _NS + s

    z16 = jnp.zeros((16,), jnp.float32)

    def zrow(i, carry):
        for j in range(8):
            rows_a[i, pl.ds(j * 16, 16)] = z16
        return carry

    lax.fori_loop(0, _CHUNK, zrow, 0)

    for k in range(_RPT // _CHUNK):
        pltpu.sync_copy(rows_a, agg_sh.at[pl.ds(s * _RPT + k * _CHUNK, _CHUNK)])

    plsc.subcore_barrier()

    pltpu.sync_copy(srcf_hbm.at[wid], src_v)
    pltpu.sync_copy(dsts_hbm.at[wid], dst_v)

    def gather(j, buf, sem):
        pltpu.async_copy(x_hbm.at[src_v.at[pl.ds(j * _CHUNK, _CHUNK)]],
                         buf, sem)

    def gwait(buf, sem):
        pltpu.make_async_copy(x_hbm.at[pl.ds(0, _CHUNK)], buf, sem).wait()

    def scat(j, buf, sem):
        pltpu.async_copy(buf, agg_sh.at[dst_v.at[j]], sem, add=True)

    def swait(buf, sem):
        pltpu.make_async_copy(buf, agg_sh.at[pl.ds(0, _CHUNK)], sem).wait()

    gather(0, rows_a, sem_a)
    gather(1, rows_b, sem_b)

    def body2(i, carry):
        j = 2 * i
        gwait(rows_a, sem_a)
        scat(j, rows_a, sem_sa)
        gwait(rows_b, sem_b)
        scat(j + 1, rows_b, sem_sb)
        swait(rows_a, sem_sa)
        gather(j + 2, rows_a, sem_a)
        swait(rows_b, sem_sb)
        gather(j + 3, rows_b, sem_b)
        return carry

    lax.fori_loop(0, (_NCHUNK - 3) // 2, body2, 0)

    gwait(rows_a, sem_a)
    scat(_NCHUNK - 3, rows_a, sem_sa)
    gwait(rows_b, sem_b)
    scat(_NCHUNK - 2, rows_b, sem_sb)
    swait(rows_a, sem_sa)
    gather(_NCHUNK - 1, rows_a, sem_a)
    swait(rows_b, sem_sb)
    gwait(rows_a, sem_a)
    scat(_NCHUNK - 1, rows_a, sem_sa)
    swait(rows_a, sem_sa)

    plsc.subcore_barrier()

    pltpu.sync_copy(agg_sh.at[pl.ds(s * _RPT, _RPT)],
                    agg_out.at[c, pl.ds(s * _RPT, _RPT)])


def _make_sc_agg():
    mesh = plsc.VectorSubcoreMesh(core_axis_name="c", subcore_axis_name="s")
    return pl.kernel(
        _sc_agg_body,
        out_type=jax.ShapeDtypeStruct((_NC, _NP, _C), jnp.float32),
        mesh=mesh,
        scratch_types=[
            pltpu.VMEM_SHARED((_NP, _C), jnp.float32),
            pltpu.VMEM((_EPT,), jnp.int32),
            pltpu.VMEM((_NCHUNK, _CHUNK), jnp.int32),
            pltpu.VMEM((_CHUNK, _C), jnp.float32),
            pltpu.VMEM((_CHUNK, _C), jnp.float32),
            pltpu.SemaphoreType.DMA,
            pltpu.SemaphoreType.DMA,
            pltpu.SemaphoreType.DMA,
            pltpu.SemaphoreType.DMA,
        ],
    )


def _sc_deg_body(dsts_hbm, deg_out, deg_sh, dst_v, ones_v, zbuf,
                 sem_a, sem_b):
    c = lax.axis_index("c")
    s = lax.axis_index("s")
    wid = c * _NS + s

    z16 = jnp.zeros((16,), jnp.float32)
    o16 = jnp.ones((16,), jnp.float32)

    def zrow(i, carry):
        for j in range(8):
            zbuf[i, pl.ds(j * 16, 16)] = z16
        return carry

    lax.fori_loop(0, _ZR, zrow, 0)

    def onerow(i, carry):
        for j in range(8):
            ones_v[i, pl.ds(j * 16, 16)] = o16
        return carry

    lax.fori_loop(0, _CHUNK, onerow, 0)

    for k in range(_RPT // _ZR):
        pltpu.sync_copy(zbuf, deg_sh.at[pl.ds(s * _RPT + k * _ZR, _ZR)])

    plsc.subcore_barrier()

    pltpu.sync_copy(dsts_hbm.at[wid], dst_v)

    def scat(j, sem):
        pltpu.async_copy(ones_v, deg_sh.at[dst_v.at[j]], sem, add=True)

    def swait(sem):
        pltpu.make_async_copy(ones_v, deg_sh.at[pl.ds(0, _CHUNK)], sem).wait()

    scat(0, sem_a)
    scat(1, sem_b)

    def body2(i, carry):
        j = 2 * i
        swait(sem_a)
        scat(j + 2, sem_a)
        swait(sem_b)
        scat(j + 3, sem_b)
        return carry

    lax.fori_loop(0, (_NCHUNK - 3) // 2, body2, 0)

    swait(sem_a)
    scat(_NCHUNK - 1, sem_a)
    swait(sem_b)
    swait(sem_a)

    plsc.subcore_barrier()

    pltpu.sync_copy(deg_sh.at[pl.ds(s * _RPT, _RPT)],
                    deg_out.at[c, pl.ds(s * _RPT, _RPT)])


def _make_sc_deg():
    mesh = plsc.VectorSubcoreMesh(core_axis_name="c", subcore_axis_name="s")
    return pl.kernel(
        _sc_deg_body,
        out_type=jax.ShapeDtypeStruct((_NC, _NP, _C), jnp.float32),
        mesh=mesh,
        scratch_types=[
            pltpu.VMEM_SHARED((_NP, _C), jnp.float32),
            pltpu.VMEM((_NCHUNK, _CHUNK), jnp.int32),
            pltpu.VMEM((_CHUNK, _C), jnp.float32),
            pltpu.VMEM((_ZR, _C), jnp.float32),
            pltpu.SemaphoreType.DMA,
            pltpu.SemaphoreType.DMA,
        ],
    )


def _mlp_body(has_res, do_norm, *refs):
    if has_res:
        (parts_ref, degp_ref, xn_ref, res_ref,
         w1a_ref, w1b_ref, b1_ref, w2_ref, b2_ref), outs = refs[:9], refs[9:]
    else:
        (parts_ref, degp_ref, xn_ref,
         w1a_ref, w1b_ref, b1_ref, w2_ref, b2_ref), outs = refs[:8], refs[8:]
        res_ref = None
    agg = parts_ref[0] + parts_ref[1]
    deg = degp_ref[0, :, 0:1] + degp_ref[1, :, 0:1]
    agg = agg / jnp.maximum(deg, 1.0)
    xn = xn_ref[...]
    u = jnp.dot(xn, w1a_ref[...], preferred_element_type=jnp.float32)
    u = u + jnp.dot(agg, w1b_ref[...], preferred_element_type=jnp.float32)
    u = jnp.maximum(u + b1_ref[...], 0.0)
    o = jnp.dot(u, w2_ref[...], preferred_element_type=jnp.float32)
    o = o + b2_ref[...]
    if has_res:
        o = o + res_ref[...]
    outs[0][...] = o
    if do_norm:
        mu = jnp.mean(o, axis=1, keepdims=True)
        sd = jnp.sqrt(jnp.mean((o - mu) ** 2, axis=1, keepdims=True)) + 1e-6
        outs[1][...] = (o - mu) / sd


_BLK = 1024


def _mlp_call(parts, degp, xn, res, W1, b1, W2, b2, has_res, do_norm):
    cout = W2.shape[1]
    grid = (_NP // _BLK,)
    in_specs = [
        pl.BlockSpec((2, _BLK, _C), lambda i: (0, i, 0)),
        pl.BlockSpec((2, _BLK, _C), lambda i: (0, i, 0)),
        pl.BlockSpec((_BLK, _C), lambda i: (i, 0)),
    ]
    args = [parts, degp, xn]
    if has_res:
        in_specs.append(pl.BlockSpec((_BLK, _C), lambda i: (i, 0)))
        args.append(res)
    in_specs += [
        pl.BlockSpec((_C, 2 * _C), lambda i: (0, 0)),
        pl.BlockSpec((_C, 2 * _C), lambda i: (0, 0)),
        pl.BlockSpec((1, 2 * _C), lambda i: (0, 0)),
        pl.BlockSpec((2 * _C, cout), lambda i: (0, 0)),
        pl.BlockSpec((1, cout), lambda i: (0, 0)),
    ]
    args += [W1[:_C], W1[_C:], b1.reshape(1, -1), W2, b2.reshape(1, -1)]
    out_shape = [jax.ShapeDtypeStruct((_NP, cout), jnp.float32)]
    out_specs = [pl.BlockSpec((_BLK, cout), lambda i: (i, 0))]
    if do_norm:
        out_shape.append(jax.ShapeDtypeStruct((_NP, cout), jnp.float32))
        out_specs.append(pl.BlockSpec((_BLK, cout), lambda i: (i, 0)))
    return pl.pallas_call(
        functools.partial(_mlp_body, has_res, do_norm),
        grid=grid,
        in_specs=in_specs,
        out_specs=out_specs,
        out_shape=out_shape,
    )(*args)


def kernel(node_features, edge_index, angles, gt_edges,
           W1_1, b1_1, W2_1, b2_1,
           W1_2, b1_2, W2_2, b2_2,
           W1_3, b1_3, W2_3, b2_3,
           W1_4, b1_4, W2_4, b2_4):
    srcf = edge_index[0].reshape(_NW, _EPT)
    dsts = edge_index[1].reshape(_NW, _NCHUNK, _CHUNK)
    xpad = jnp.pad(node_features, ((0, _NP - _N), (0, 0)))

    sc_agg = _make_sc_agg()
    degp = _make_sc_deg()(dsts)

    parts1 = sc_agg(xpad, srcf, dsts)
    h1, xn2 = _mlp_call(parts1, degp, xpad, None,
                        W1_1, b1_1, W2_1, b2_1, False, True)
    parts2 = sc_agg(xn2, srcf, dsts)
    h2, xn3 = _mlp_call(parts2, degp, xn2, h1,
                        W1_2, b1_2, W2_2, b2_2, True, True)
    parts3 = sc_agg(xn3, srcf, dsts)
    h3, xn4 = _mlp_call(parts3, degp, xn3, h2,
                        W1_3, b1_3, W2_3, b2_3, True, True)
    parts4 = sc_agg(xn4, srcf, dsts)
    (out,) = _mlp_call(parts4, degp, xn4, None,
                       W1_4, b1_4, W2_4, b2_4, False, False)
    return (out[:_N], jnp.zeros((1,), jnp.float32))


# speedup vs baseline: 1.2303x; 1.2303x over previous
import functools

import jax
import jax.numpy as jnp
from jax import lax
from jax.experimental import pallas as pl
from jax.experimental.pallas import tpu as pltpu
from jax.experimental.pallas import tpu_sc as plsc

_N = 10000
_NP = 10240
_C = 128
_E = 320000
_NC = 2
_NS = 16
_NW = _NC * _NS
_EPT = _E // _NW
_CHUNK = 80
_NCHUNK = _EPT // _CHUNK
_RPT = _NP // _NS
_ZR = 64


def _sc_agg_body(with_deg, *refs):
    if with_deg:
        (x_hbm, srcf_hbm, dsts_hbm, agg_out, deg_out,
         agg_sh, src_v, dst_v, rows_a, rows_b, sem_a, sem_b) = refs
    else:
        (x_hbm, srcf_hbm, dsts_hbm, agg_out,
         agg_sh, src_v, dst_v, rows_a, rows_b, sem_a, sem_b) = refs
    c = lax.axis_index("c")
    s = lax.axis_index("s")
    wid = c * _NS + s

    z16 = jnp.zeros((16,), jnp.float32)

    def zrow(i, carry):
        for j in range(8):
            rows_a[i, pl.ds(j * 16, 16)] = z16
        return carry

    lax.fori_loop(0, _CHUNK, zrow, 0)

    for k in range(_RPT // _CHUNK):
        pltpu.sync_copy(rows_a, agg_sh.at[pl.ds(s * _RPT + k * _CHUNK, _CHUNK)])

    plsc.subcore_barrier()

    pltpu.sync_copy(srcf_hbm.at[wid], src_v)
    pltpu.sync_copy(dsts_hbm.at[wid], dst_v)

    def gather(j, buf, sem):
        pltpu.async_copy(x_hbm.at[src_v.at[pl.ds(j * _CHUNK, _CHUNK)]],
                         buf, sem)

    def gwait(buf, sem):
        pltpu.make_async_copy(x_hbm.at[pl.ds(0, _CHUNK)], buf, sem).wait()

    gather(0, rows_a, sem_a)

    def body2(i, carry):
        j = 2 * i
        gather(j + 1, rows_b, sem_b)
        gwait(rows_a, sem_a)
        pltpu.sync_copy(rows_a, agg_sh.at[dst_v.at[j]], add=True)
        gather(j + 2, rows_a, sem_a)
        gwait(rows_b, sem_b)
        pltpu.sync_copy(rows_b, agg_sh.at[dst_v.at[j + 1]], add=True)
        return carry

    lax.fori_loop(0, (_NCHUNK - 1) // 2, body2, 0)

    gwait(rows_a, sem_a)
    pltpu.sync_copy(rows_a, agg_sh.at[dst_v.at[_NCHUNK - 1]], add=True)

    plsc.subcore_barrier()

    pltpu.sync_copy(agg_sh.at[pl.ds(s * _RPT, _RPT)],
                    agg_out.at[c, pl.ds(s * _RPT, _RPT)])

    if with_deg:
        def zrow2(i, carry):
            for j in range(8):
                rows_a[i, pl.ds(j * 16, 16)] = z16
            return carry

        lax.fori_loop(0, _CHUNK, zrow2, 0)
        for k in range(_RPT // _CHUNK):
            pltpu.sync_copy(rows_a,
                            agg_sh.at[pl.ds(s * _RPT + k * _CHUNK, _CHUNK)])

        o16 = jnp.ones((16,), jnp.float32)

        def orow(i, carry):
            for j in range(8):
                rows_a[i, pl.ds(j * 16, 16)] = o16
            return carry

        lax.fori_loop(0, _CHUNK, orow, 0)

        plsc.subcore_barrier()

        def dchunk(j, carry):
            pltpu.sync_copy(rows_a, agg_sh.at[dst_v.at[j]], add=True)
            return carry

        lax.fori_loop(0, _NCHUNK, dchunk, 0)

        plsc.subcore_barrier()

        pltpu.sync_copy(agg_sh.at[pl.ds(s * _RPT, _RPT)],
                        deg_out.at[c, pl.ds(s * _RPT, _RPT)])


def _make_sc_agg(with_deg=False):
    mesh = plsc.VectorSubcoreMesh(core_axis_name="c", subcore_axis_name="s")
    out_type = [jax.ShapeDtypeStruct((_NC, _NP, _C), jnp.float32)]
    if with_deg:
        out_type.append(jax.ShapeDtypeStruct((_NC, _NP, _C), jnp.float32))
    return pl.kernel(
        functools.partial(_sc_agg_body, with_deg),
        out_type=tuple(out_type),
        mesh=mesh,
        scratch_types=[
            pltpu.VMEM_SHARED((_NP, _C), jnp.float32),
            pltpu.VMEM((_EPT,), jnp.int32),
            pltpu.VMEM((_NCHUNK, _CHUNK), jnp.int32),
            pltpu.VMEM((_CHUNK, _C), jnp.float32),
            pltpu.VMEM((_CHUNK, _C), jnp.float32),
            pltpu.SemaphoreType.DMA,
            pltpu.SemaphoreType.DMA,
        ],
    )


def _mlp_body(has_res, do_norm, *refs):
    if has_res:
        (parts_ref, degp_ref, xn_ref, res_ref,
         w1a_ref, w1b_ref, b1_ref, w2_ref, b2_ref), outs = refs[:9], refs[9:]
    else:
        (parts_ref, degp_ref, xn_ref,
         w1a_ref, w1b_ref, b1_ref, w2_ref, b2_ref), outs = refs[:8], refs[8:]
        res_ref = None
    agg = parts_ref[0] + parts_ref[1]
    deg = degp_ref[0, :, 0:1] + degp_ref[1, :, 0:1]
    agg = agg / jnp.maximum(deg, 1.0)
    xn = xn_ref[...]
    u = jnp.dot(xn, w1a_ref[...], preferred_element_type=jnp.float32)
    u = u + jnp.dot(agg, w1b_ref[...], preferred_element_type=jnp.float32)
    u = jnp.maximum(u + b1_ref[...], 0.0)
    o = jnp.dot(u, w2_ref[...], preferred_element_type=jnp.float32)
    o = o + b2_ref[...]
    if has_res:
        o = o + res_ref[...]
    outs[0][...] = o
    if do_norm:
        mu = jnp.mean(o, axis=1, keepdims=True)
        sd = jnp.sqrt(jnp.mean((o - mu) ** 2, axis=1, keepdims=True)) + 1e-6
        outs[1][...] = (o - mu) / sd


_BLK = 1024


def _mlp_call(parts, degp, xn, res, W1, b1, W2, b2, has_res, do_norm):
    cout = W2.shape[1]
    grid = (_NP // _BLK,)
    in_specs = [
        pl.BlockSpec((2, _BLK, _C), lambda i: (0, i, 0)),
        pl.BlockSpec((2, _BLK, _C), lambda i: (0, i, 0)),
        pl.BlockSpec((_BLK, _C), lambda i: (i, 0)),
    ]
    args = [parts, degp, xn]
    if has_res:
        in_specs.append(pl.BlockSpec((_BLK, _C), lambda i: (i, 0)))
        args.append(res)
    in_specs += [
        pl.BlockSpec((_C, 2 * _C), lambda i: (0, 0)),
        pl.BlockSpec((_C, 2 * _C), lambda i: (0, 0)),
        pl.BlockSpec((1, 2 * _C), lambda i: (0, 0)),
        pl.BlockSpec((2 * _C, cout), lambda i: (0, 0)),
        pl.BlockSpec((1, cout), lambda i: (0, 0)),
    ]
    args += [W1[:_C], W1[_C:], b1.reshape(1, -1), W2, b2.reshape(1, -1)]
    out_shape = [jax.ShapeDtypeStruct((_NP, cout), jnp.float32)]
    out_specs = [pl.BlockSpec((_BLK, cout), lambda i: (i, 0))]
    if do_norm:
        out_shape.append(jax.ShapeDtypeStruct((_NP, cout), jnp.float32))
        out_specs.append(pl.BlockSpec((_BLK, cout), lambda i: (i, 0)))
    return pl.pallas_call(
        functools.partial(_mlp_body, has_res, do_norm),
        grid=grid,
        in_specs=in_specs,
        out_specs=out_specs,
        out_shape=out_shape,
    )(*args)


def kernel(node_features, edge_index, angles, gt_edges,
           W1_1, b1_1, W2_1, b2_1,
           W1_2, b1_2, W2_2, b2_2,
           W1_3, b1_3, W2_3, b2_3,
           W1_4, b1_4, W2_4, b2_4):
    srcf = edge_index[0].reshape(_NW, _EPT)
    dsts = edge_index[1].reshape(_NW, _NCHUNK, _CHUNK)
    xpad = jnp.pad(node_features, ((0, _NP - _N), (0, 0)))

    sc_agg = _make_sc_agg(False)

    parts1, degp = _make_sc_agg(True)(xpad, srcf, dsts)
    h1, xn2 = _mlp_call(parts1, degp, xpad, None,
                        W1_1, b1_1, W2_1, b2_1, False, True)
    (parts2,) = sc_agg(xn2, srcf, dsts)
    h2, xn3 = _mlp_call(parts2, degp, xn2, h1,
                        W1_2, b1_2, W2_2, b2_2, True, True)
    (parts3,) = sc_agg(xn3, srcf, dsts)
    h3, xn4 = _mlp_call(parts3, degp, xn3, h2,
                        W1_3, b1_3, W2_3, b2_3, True, True)
    (parts4,) = sc_agg(xn4, srcf, dsts)
    (out,) = _mlp_call(parts4, degp, xn4, None,
                       W1_4, b1_4, W2_4, b2_4, False, False)
    return (out[:_N], jnp.zeros((1,), jnp.float32))
